# Initial kernel scaffold; baseline (speedup 1.0000x reference)
import jax
import jax.numpy as jnp
from jax.experimental import pallas as pl


def kernel(X, table):
    raise NotImplementedError("write your pallas kernel here")


# speedup vs baseline: 1.5678x; 1.5678x over previous
import functools

import jax
import jax.numpy as jnp
from jax import lax
from jax.experimental import pallas as pl
from jax.experimental.pallas import tpu as pltpu
from jax.experimental.pallas import tpu_sc as plsc

_BATCH = 16384
_FIELDS = 26
_DIM = 32
_ROWS = _BATCH * _FIELDS
_NW = 32
_RPW = _ROWS // _NW
_CHUNK = 1664
_NCH = _RPW // _CHUNK


@functools.partial(
    pl.kernel,
    mesh=plsc.VectorSubcoreMesh(core_axis_name="c", subcore_axis_name="s"),
    out_type=jax.ShapeDtypeStruct((_ROWS, _DIM), jnp.float32),
    scratch_types=[
        pltpu.VMEM((_RPW,), jnp.int32),
        pltpu.VMEM((_CHUNK, _DIM), jnp.float32),
        pltpu.SemaphoreType.DMA,
    ],
    compiler_params=pltpu.CompilerParams(use_tc_tiling_on_sc=False),
)
def _gather_kernel(table, idx, out, idx_v, rows_v, sem):
    w = lax.axis_index("s") * 2 + lax.axis_index("c")
    base = pl.multiple_of(w * _RPW, 8)
    pltpu.sync_copy(idx.at[pl.ds(base, _RPW)], idx_v)
    for c in range(_NCH):
        pltpu.async_copy(
            table.at[idx_v.at[pl.ds(c * _CHUNK, _CHUNK)]], rows_v, sem
        ).wait()
        pltpu.sync_copy(rows_v, out.at[pl.ds(base + c * _CHUNK, _CHUNK)])


def kernel(X, table):
    idx = X.reshape(_ROWS)
    out = _gather_kernel(table, idx)
    return out.reshape(_BATCH, _FIELDS, _DIM)


# speedup vs baseline: 1.5757x; 1.0051x over previous
import functools

import jax
import jax.numpy as jnp
from jax import lax
from jax.experimental import pallas as pl
from jax.experimental.pallas import tpu as pltpu
from jax.experimental.pallas import tpu_sc as plsc

_BATCH = 16384
_FIELDS = 26
_DIM = 32
_ROWS = _BATCH * _FIELDS
_NW = 32
_RPW = _ROWS // _NW
_CHUNK = 1664
_NCH = _RPW // _CHUNK


@functools.partial(
    pl.kernel,
    mesh=plsc.VectorSubcoreMesh(core_axis_name="c", subcore_axis_name="s"),
    out_type=jax.ShapeDtypeStruct((_ROWS, _DIM), jnp.float32),
    scratch_types=[
        pltpu.VMEM((_RPW,), jnp.int32),
        pltpu.VMEM((2, _CHUNK, _DIM), jnp.float32),
        pltpu.SemaphoreType.DMA,
        pltpu.SemaphoreType.DMA,
        pltpu.SemaphoreType.DMA,
        pltpu.SemaphoreType.DMA,
    ],
    compiler_params=pltpu.CompilerParams(use_tc_tiling_on_sc=False),
)
def _gather_kernel(table, idx, out, idx_v, rows_v, sem_g0, sem_g1, sem_o0, sem_o1):
    w = lax.axis_index("s") * 2 + lax.axis_index("c")
    base = pl.multiple_of(w * _RPW, 8)
    pltpu.sync_copy(idx.at[pl.ds(base, _RPW)], idx_v)
    sems_g = (sem_g0, sem_g1)
    sems_o = (sem_o0, sem_o1)

    def gather(c):
        b = c % 2
        return pltpu.make_async_copy(
            table.at[idx_v.at[pl.ds(c * _CHUNK, _CHUNK)]], rows_v.at[b], sems_g[b]
        )

    def outcp(c):
        b = c % 2
        return pltpu.make_async_copy(
            rows_v.at[b], out.at[pl.ds(base + c * _CHUNK, _CHUNK)], sems_o[b]
        )

    gather(0).start()
    for c in range(_NCH):
        if c + 1 < _NCH:
            if c - 1 >= 0:
                outcp(c - 1).wait()
            gather(c + 1).start()
        gather(c).wait()
        outcp(c).start()
    outcp(_NCH - 2).wait()
    outcp(_NCH - 1).wait()


def kernel(X, table):
    idx = X.reshape(_ROWS)
    out = _gather_kernel(table, idx)
    return out.reshape(_BATCH, _FIELDS, _DIM)


# speedup vs baseline: 1.5763x; 1.0003x over previous
import functools

import jax
import jax.numpy as jnp
from jax import lax
from jax.experimental import pallas as pl
from jax.experimental.pallas import tpu as pltpu
from jax.experimental.pallas import tpu_sc as plsc

_NUM_ROWS_TBL = 1000000
_BATCH = 16384
_FIELDS = 26
_DIM = 32
_ROWS = _BATCH * _FIELDS
_NW = 32
_RPW = _ROWS // _NW
_CHUNK = 1664
_NCH = _RPW // _CHUNK


@functools.partial(
    pl.kernel,
    mesh=plsc.VectorSubcoreMesh(core_axis_name="c", subcore_axis_name="s"),
    out_type=jax.ShapeDtypeStruct((_ROWS, _DIM), jnp.float32),
    scratch_types=[
        pltpu.VMEM((_RPW,), jnp.int32),
        pltpu.VMEM((2, _CHUNK, _DIM), jnp.float32),
        pltpu.SemaphoreType.DMA,
        pltpu.SemaphoreType.DMA,
        pltpu.SemaphoreType.DMA,
        pltpu.SemaphoreType.DMA,
    ],
    compiler_params=pltpu.CompilerParams(use_tc_tiling_on_sc=False),
)
def _gather_kernel(table, idx, out, idx_v, rows_v, sem_g0, sem_g1, sem_o0, sem_o1):
    w = lax.axis_index("s") * 2 + lax.axis_index("c")
    base = pl.multiple_of(w * _RPW, 8)
    pltpu.sync_copy(idx.at[pl.ds(base, _RPW)], idx_v)
    sems_g = (sem_g0, sem_g1)
    sems_o = (sem_o0, sem_o1)

    def gather(c):
        b = c % 2
        return pltpu.make_async_copy(
            table.at[idx_v.at[pl.ds(c * _CHUNK, _CHUNK)]], rows_v.at[b], sems_g[b]
        )

    def outcp(c):
        b = c % 2
        return pltpu.make_async_copy(
            rows_v.at[b], out.at[pl.ds(base + c * _CHUNK, _CHUNK)], sems_o[b]
        )

    gather(0).start()
    for c in range(_NCH):
        if c + 1 < _NCH:
            if c - 1 >= 0:
                outcp(c - 1).wait()
            gather(c + 1).start()
        gather(c).wait()
        outcp(c).start()
    outcp(_NCH - 2).wait()
    outcp(_NCH - 1).wait()


def kernel(X, table):
    idx = X.reshape(_ROWS)
    tbl = jax.lax.optimization_barrier(table.reshape(_NUM_ROWS_TBL // 4, _DIM * 4))
    out = _gather_kernel(tbl.reshape(_NUM_ROWS_TBL, _DIM), idx)
    return out.reshape(_BATCH, _FIELDS, _DIM)


# speedup vs baseline: 1.5770x; 1.0005x over previous
import functools

import jax
import jax.numpy as jnp
from jax import lax
from jax.experimental import pallas as pl
from jax.experimental.pallas import tpu as pltpu
from jax.experimental.pallas import tpu_sc as plsc

_NTBL = 1000000
_BATCH = 16384
_FIELDS = 26
_DIM = 32
_ROWS = _BATCH * _FIELDS
_NW = 32
_RPW = _ROWS // _NW
_CHUNK = 1664
_NCH = _RPW // _CHUNK


@functools.partial(
    pl.kernel,
    mesh=plsc.VectorSubcoreMesh(core_axis_name="c", subcore_axis_name="s"),
    out_type=jax.ShapeDtypeStruct((_ROWS, _DIM), jnp.float32),
    scratch_types=[
        pltpu.VMEM((_RPW,), jnp.int32),
        pltpu.VMEM((2, _CHUNK, _DIM), jnp.float32),
        pltpu.SemaphoreType.DMA,
        pltpu.SemaphoreType.DMA,
        pltpu.SemaphoreType.DMA,
        pltpu.SemaphoreType.DMA,
    ],
    compiler_params=pltpu.CompilerParams(use_tc_tiling_on_sc=False),
)
def _gather_kernel(table, idx, out, idx_v, rows_v, sem_g0, sem_g1, sem_o0, sem_o1):
    w = lax.axis_index("s") * 2 + lax.axis_index("c")
    base = pl.multiple_of(w * _RPW, 8)
    pltpu.sync_copy(idx.at[pl.ds(base, _RPW)], idx_v)
    sems_g = (sem_g0, sem_g1)
    sems_o = (sem_o0, sem_o1)

    def gather(c):
        b = c % 2
        return pltpu.make_async_copy(
            table.at[idx_v.at[pl.ds(c * _CHUNK, _CHUNK)]], rows_v.at[b], sems_g[b]
        )

    def outcp(c):
        b = c % 2
        return pltpu.make_async_copy(
            rows_v.at[b], out.at[pl.ds(base + c * _CHUNK, _CHUNK)], sems_o[b]
        )

    gather(0).start()
    for c in range(_NCH):
        if c + 1 < _NCH:
            if c - 1 >= 0:
                outcp(c - 1).wait()
            gather(c + 1).start()
        gather(c).wait()
        outcp(c).start()
    outcp(_NCH - 2).wait()
    outcp(_NCH - 1).wait()


def kernel(X, table):
    idx = X.reshape(_ROWS)
    tbl = jax.lax.optimization_barrier(table.reshape(_NTBL * _DIM))
    out = _gather_kernel(tbl.reshape(_NTBL, _DIM), idx)
    return out.reshape(_BATCH, _FIELDS, _DIM)
